# Initial kernel scaffold; baseline (speedup 1.0000x reference)
import jax
import jax.numpy as jnp
from jax.experimental import pallas as pl


def kernel(x, edge_index, W_embed, b_embed, conv_W, conv_b, bn_gamma, bn_beta, bn_mean, bn_var, W_out, b_out):
    raise NotImplementedError("write your pallas kernel here")


# speedup vs baseline: 6.7355x; 6.7355x over previous
import functools

import jax
import jax.numpy as jnp
from jax import lax
from jax.experimental import pallas as pl
from jax.experimental.pallas import tpu as pltpu
from jax.experimental.pallas import tpu_sc as plsc

_N = 10000
_E = 320000
_H = 128
_C = 40

_CH = 80
_NW = 32
_ROWS_PER_W = _E // _NW // _CH
_NP = 10240


def _sc_agg_body(h, srcm, dstm, zbig, zsmall,
                 agg_out, cnt_out,
                 idx_s, idx_d, rows, ones, agg_sh, cnt_sh, sem):
    cid = lax.axis_index("c")
    sid = lax.axis_index("s")
    wid = cid * 16 + sid

    pltpu.sync_copy(zbig.at[pl.ds(sid * 640, 640)],
                    agg_sh.at[pl.ds(sid * 640, 640)])
    pltpu.sync_copy(zsmall.at[pl.ds(sid * 640, 640)],
                    cnt_sh.at[pl.ds(sid * 640, 640)])

    for i in range(_CH // 16):
        ones[pl.ds(i * 16, 16)] = jnp.ones((16,), jnp.float32)

    pltpu.sync_copy(srcm.at[wid], idx_s)
    pltpu.sync_copy(dstm.at[wid], idx_d)

    plsc.subcore_barrier()

    def chunk(c, _):
        pltpu.async_copy(h.at[idx_s.at[c]], rows, sem).wait()
        pltpu.sync_copy(rows, agg_sh.at[idx_d.at[c]], add=True)
        pltpu.sync_copy(ones, cnt_sh.at[idx_d.at[c]], add=True)
        return ()

    lax.fori_loop(0, _ROWS_PER_W, chunk, ())

    plsc.subcore_barrier()

    pltpu.sync_copy(agg_sh.at[pl.ds(sid * 640, 640)],
                    agg_out.at[cid].at[pl.ds(sid * 640, 640)])
    pltpu.sync_copy(cnt_sh.at[pl.ds(sid * 640, 640)],
                    cnt_out.at[cid].at[pl.ds(sid * 640, 640)])


_sc_aggregate = pl.kernel(
    _sc_agg_body,
    out_type=[
        jax.ShapeDtypeStruct((2, _NP, _H), jnp.float32),
        jax.ShapeDtypeStruct((2, _NP), jnp.float32),
    ],
    mesh=plsc.VectorSubcoreMesh(core_axis_name="c", subcore_axis_name="s"),
    scratch_types=[
        pltpu.VMEM((_ROWS_PER_W, _CH), jnp.int32),
        pltpu.VMEM((_ROWS_PER_W, _CH), jnp.int32),
        pltpu.VMEM((_CH, _H), jnp.float32),
        pltpu.VMEM((_CH,), jnp.float32),
        pltpu.VMEM_SHARED((_NP, _H), jnp.float32),
        pltpu.VMEM_SHARED((_NP,), jnp.float32),
        pltpu.SemaphoreType.DMA,
    ],
)


_BN = 1000


def _embed_body(x_ref, w_ref, b_ref, o_ref):
    o_ref[...] = (jnp.dot(x_ref[...], w_ref[...],
                          preferred_element_type=jnp.float32) + b_ref[...])


def _embed(x, w, b):
    return pl.pallas_call(
        _embed_body,
        grid=(_N // _BN,),
        in_specs=[
            pl.BlockSpec((_BN, _H), lambda i: (i, 0)),
            pl.BlockSpec((_H, _H), lambda i: (0, 0)),
            pl.BlockSpec((1, _H), lambda i: (0, 0)),
        ],
        out_specs=pl.BlockSpec((_BN, _H), lambda i: (i, 0)),
        out_shape=jax.ShapeDtypeStruct((_N, _H), jnp.float32),
    )(x, w, b)


def _layer_body(p0_ref, p1_ref, c0_ref, c1_ref, h_ref, w_ref, b_ref, o_ref):
    deg = jnp.maximum(c0_ref[...] + c1_ref[...], 1.0)
    a = (p0_ref[...] + p1_ref[...]) / deg
    y = jnp.dot(a, w_ref[...], preferred_element_type=jnp.float32) + b_ref[...]
    o_ref[...] = jnp.maximum(y, 0.0) + h_ref[...]


def _layer(p0, p1, c0, c1, h, w, b):
    return pl.pallas_call(
        _layer_body,
        grid=(_N // _BN,),
        in_specs=[
            pl.BlockSpec((_BN, _H), lambda i: (i, 0)),
            pl.BlockSpec((_BN, _H), lambda i: (i, 0)),
            pl.BlockSpec((_BN, 1), lambda i: (i, 0)),
            pl.BlockSpec((_BN, 1), lambda i: (i, 0)),
            pl.BlockSpec((_BN, _H), lambda i: (i, 0)),
            pl.BlockSpec((_H, _H), lambda i: (0, 0)),
            pl.BlockSpec((1, _H), lambda i: (0, 0)),
        ],
        out_specs=pl.BlockSpec((_BN, _H), lambda i: (i, 0)),
        out_shape=jax.ShapeDtypeStruct((_N, _H), jnp.float32),
    )(p0, p1, c0, c1, h, w, b)


def _head_body(h_ref, w_ref, b_ref, o_ref):
    y = (jnp.dot(h_ref[...], w_ref[...], preferred_element_type=jnp.float32)
         + b_ref[...])
    m = jnp.max(y, axis=-1, keepdims=True)
    lse = jnp.log(jnp.sum(jnp.exp(y - m), axis=-1, keepdims=True)) + m
    o_ref[...] = y - lse


def _head(h, w, b):
    return pl.pallas_call(
        _head_body,
        grid=(_N // _BN,),
        in_specs=[
            pl.BlockSpec((_BN, _H), lambda i: (i, 0)),
            pl.BlockSpec((_H, _C), lambda i: (0, 0)),
            pl.BlockSpec((1, _C), lambda i: (0, 0)),
        ],
        out_specs=pl.BlockSpec((_BN, _C), lambda i: (i, 0)),
        out_shape=jax.ShapeDtypeStruct((_N, _C), jnp.float32),
    )(h, w, b)


@jax.jit
def kernel(x, edge_index, W_embed, b_embed, conv_W, conv_b,
           bn_gamma, bn_beta, bn_mean, bn_var, W_out, b_out):
    srcm = edge_index[0].reshape(_NW, _ROWS_PER_W, _CH)
    dstm = edge_index[1].reshape(_NW, _ROWS_PER_W, _CH)
    zbig = jnp.zeros((_NP, _H), jnp.float32)
    zsmall = jnp.zeros((_NP,), jnp.float32)

    s = bn_gamma / jnp.sqrt(bn_var + 1e-5)
    w_fold = conv_W * s[:, None, :]
    b_fold = conv_b * s + bn_beta - bn_mean * s

    h = _embed(x, W_embed, b_embed.reshape(1, _H))

    for i in range(3):
        agg, cnt = _sc_aggregate(h, srcm, dstm, zbig, zsmall)
        c0 = cnt[0, :_N].reshape(_N, 1)
        c1 = cnt[1, :_N].reshape(_N, 1)
        h = _layer(agg[0, :_N], agg[1, :_N], c0, c1, h,
                   w_fold[i], b_fold[i].reshape(1, _H))

    return _head(h, W_out, b_out.reshape(1, _C))


# speedup vs baseline: 9.8945x; 1.4690x over previous
import jax
import jax.numpy as jnp
from jax import lax
from jax.experimental import pallas as pl
from jax.experimental.pallas import tpu as pltpu
from jax.experimental.pallas import tpu_sc as plsc

_N = 10000
_E = 320000
_H = 128
_C = 40

_CH = 80
_NW = 32
_SB = 25
_NSB = _E // _NW // _CH // _SB
_NP = 10240


def _sc_agg_body(h, srcm, dstm, zbig, zsmall,
                 agg_out, cnt_out,
                 idx_s, idx_d, rows0, rows1, ones, agg_sh, cnt_sh,
                 sem0, sem1):
    cid = lax.axis_index("c")
    sid = lax.axis_index("s")
    wid = cid * 16 + sid

    pltpu.sync_copy(zbig.at[pl.ds(sid * 640, 640)],
                    agg_sh.at[pl.ds(sid * 640, 640)])

    pltpu.sync_copy(zsmall.at[pl.ds(sid * 640, 640)],
                    cnt_sh.at[pl.ds(sid * 640, 640)])

    for i in range(_CH // 16):
        ones[pl.ds(i * 16, 16)] = jnp.ones((16,), jnp.float32)

    plsc.subcore_barrier()

    def gather(c, rows, sem):
        pltpu.async_copy(h.at[idx_s.at[c]], rows, sem)

    def gwait(rows, sem):
        pltpu.make_async_copy(h.at[idx_s.at[0]], rows, sem).wait()

    def scatter(c, rows):
        pltpu.sync_copy(rows, agg_sh.at[idx_d.at[c]], add=True)
        pltpu.sync_copy(ones, cnt_sh.at[idx_d.at[c]], add=True)

    def superblock(sb, _):
        pltpu.sync_copy(srcm.at[wid].at[sb], idx_s)
        pltpu.sync_copy(dstm.at[wid].at[sb], idx_d)

        gather(0, rows0, sem0)

        def pair(i, __):
            c = i * 2
            gather(c + 1, rows1, sem1)
            gwait(rows0, sem0)
            scatter(c, rows0)
            gather(c + 2, rows0, sem0)
            gwait(rows1, sem1)
            scatter(c + 1, rows1)
            return ()

        lax.fori_loop(0, (_SB - 1) // 2, pair, ())
        gwait(rows0, sem0)
        scatter(_SB - 1, rows0)
        return ()

    lax.fori_loop(0, _NSB, superblock, ())

    plsc.subcore_barrier()

    pltpu.sync_copy(agg_sh.at[pl.ds(sid * 640, 640)],
                    agg_out.at[cid].at[pl.ds(sid * 640, 640)])

    pltpu.sync_copy(cnt_sh.at[pl.ds(sid * 640, 640)],
                    cnt_out.at[cid].at[pl.ds(sid * 640, 640)])


_sc_aggregate = pl.kernel(
    _sc_agg_body,
    out_type=[
        jax.ShapeDtypeStruct((2, _NP, _H), jnp.float32),
        jax.ShapeDtypeStruct((2, _NP), jnp.float32),
    ],
    mesh=plsc.VectorSubcoreMesh(core_axis_name="c", subcore_axis_name="s"),
    scratch_types=[
        pltpu.VMEM((_SB, _CH), jnp.int32),
        pltpu.VMEM((_SB, _CH), jnp.int32),
        pltpu.VMEM((_CH, _H), jnp.float32),
        pltpu.VMEM((_CH, _H), jnp.float32),
        pltpu.VMEM((_CH,), jnp.float32),
        pltpu.VMEM_SHARED((_NP, _H), jnp.float32),
        pltpu.VMEM_SHARED((_NP,), jnp.float32),
        pltpu.SemaphoreType.DMA,
        pltpu.SemaphoreType.DMA,
    ],
)


_BN = 1000


def _embed_body(x_ref, w_ref, b_ref, o_ref):
    o_ref[...] = (jnp.dot(x_ref[...], w_ref[...],
                          preferred_element_type=jnp.float32) + b_ref[...])


def _embed(x, w, b):
    return pl.pallas_call(
        _embed_body,
        grid=(_N // _BN,),
        in_specs=[
            pl.BlockSpec((_BN, _H), lambda i: (i, 0)),
            pl.BlockSpec((_H, _H), lambda i: (0, 0)),
            pl.BlockSpec((1, _H), lambda i: (0, 0)),
        ],
        out_specs=pl.BlockSpec((_BN, _H), lambda i: (i, 0)),
        out_shape=jax.ShapeDtypeStruct((_N, _H), jnp.float32),
    )(x, w, b)


def _layer_body(p0_ref, p1_ref, c0_ref, c1_ref, h_ref, w_ref, b_ref, o_ref):
    deg = jnp.maximum(c0_ref[...] + c1_ref[...], 1.0)
    a = (p0_ref[...] + p1_ref[...]) / deg
    y = jnp.dot(a, w_ref[...], preferred_element_type=jnp.float32) + b_ref[...]
    o_ref[...] = jnp.maximum(y, 0.0) + h_ref[...]


def _layer(p0, p1, c0, c1, h, w, b):
    return pl.pallas_call(
        _layer_body,
        grid=(_N // _BN,),
        in_specs=[
            pl.BlockSpec((_BN, _H), lambda i: (i, 0)),
            pl.BlockSpec((_BN, _H), lambda i: (i, 0)),
            pl.BlockSpec((_BN, 1), lambda i: (i, 0)),
            pl.BlockSpec((_BN, 1), lambda i: (i, 0)),
            pl.BlockSpec((_BN, _H), lambda i: (i, 0)),
            pl.BlockSpec((_H, _H), lambda i: (0, 0)),
            pl.BlockSpec((1, _H), lambda i: (0, 0)),
        ],
        out_specs=pl.BlockSpec((_BN, _H), lambda i: (i, 0)),
        out_shape=jax.ShapeDtypeStruct((_N, _H), jnp.float32),
    )(p0, p1, c0, c1, h, w, b)


def _head_body(h_ref, w_ref, b_ref, o_ref):
    y = (jnp.dot(h_ref[...], w_ref[...], preferred_element_type=jnp.float32)
         + b_ref[...])
    m = jnp.max(y, axis=-1, keepdims=True)
    lse = jnp.log(jnp.sum(jnp.exp(y - m), axis=-1, keepdims=True)) + m
    o_ref[...] = y - lse


def _head(h, w, b):
    return pl.pallas_call(
        _head_body,
        grid=(_N // _BN,),
        in_specs=[
            pl.BlockSpec((_BN, _H), lambda i: (i, 0)),
            pl.BlockSpec((_H, _C), lambda i: (0, 0)),
            pl.BlockSpec((1, _C), lambda i: (0, 0)),
        ],
        out_specs=pl.BlockSpec((_BN, _C), lambda i: (i, 0)),
        out_shape=jax.ShapeDtypeStruct((_N, _C), jnp.float32),
    )(h, w, b)


@jax.jit
def kernel(x, edge_index, W_embed, b_embed, conv_W, conv_b,
           bn_gamma, bn_beta, bn_mean, bn_var, W_out, b_out):
    srcm = edge_index[0].reshape(_NW, _NSB, _SB, _CH)
    dstm = edge_index[1].reshape(_NW, _NSB, _SB, _CH)
    zbig = jnp.zeros((_NP, _H), jnp.float32)
    zsmall = jnp.zeros((_NP,), jnp.float32)

    s = bn_gamma / jnp.sqrt(bn_var + 1e-5)
    w_fold = conv_W * s[:, None, :]
    b_fold = conv_b * s + bn_beta - bn_mean * s

    h = _embed(x, W_embed, b_embed.reshape(1, _H))

    for i in range(3):
        agg, cnt = _sc_aggregate(h, srcm, dstm, zbig, zsmall)
        h = _layer(agg[0, :_N], agg[1, :_N],
                   cnt[0, :_N].reshape(_N, 1), cnt[1, :_N].reshape(_N, 1), h,
                   w_fold[i], b_fold[i].reshape(1, _H))

    return _head(h, W_out, b_out.reshape(1, _C))
